# baseline (speedup 1.0000x reference)
import functools

import jax
import jax.numpy as jnp
from jax import lax
from jax.experimental import pallas as pl
from jax.experimental.pallas import tpu as pltpu
from jax.experimental.pallas import tpu_sc as plsc

V, DW, T, DT, H = 100000, 128, 512, 64, 256
B, L = 4096, 50
N_ROWS = B * L

_NC, _NS = 2, 16
_NW = _NC * _NS
_CHUNK = 128
_ROWS_PER_W = N_ROWS // _NW
_CHUNKS_PER_W = _ROWS_PER_W // _CHUNK


def _sc_gather_body(chunks_per_w, ids_hbm, table_hbm, out_hbm, idx_all, rows, sem):
    wid = lax.axis_index("s") * _NC + lax.axis_index("c")
    chunk_base = wid * chunks_per_w
    pltpu.sync_copy(ids_hbm.at[wid], idx_all)

    def chunk(j, carry):
        pltpu.async_copy(table_hbm.at[idx_all.at[j]], rows, sem).wait()
        pltpu.sync_copy(rows, out_hbm.at[pl.ds((chunk_base + j) * _CHUNK, _CHUNK)])
        return carry

    lax.fori_loop(0, chunks_per_w, chunk, 0)


def _sc_gather(word_ids_flat, word_table, n_rows, chunks_per_w):
    ids2d = word_ids_flat.reshape(_NW, chunks_per_w, _CHUNK)
    mesh = plsc.VectorSubcoreMesh(core_axis_name="c", subcore_axis_name="s")
    k = functools.partial(
        pl.kernel,
        mesh=mesh,
        out_type=jax.ShapeDtypeStruct((n_rows, DW), jnp.float32),
        scratch_types=[
            pltpu.VMEM((chunks_per_w, _CHUNK), jnp.int32),
            pltpu.VMEM((_CHUNK, DW), jnp.float32),
            pltpu.SemaphoreType.DMA,
        ],
    )(functools.partial(_sc_gather_body, chunks_per_w))
    return k(ids2d, word_table)


def _tc_body(we_ref, wid_ref, tid_ref, tt_ref, wtt_ref, wwt_ref,
             bw_ref, bt_ref, g_ref, b_ref, out_ref, e_ref,
             wtc_ref, pc_ref, bcc_ref):
    bb = tid_ref.shape[0]
    rows = bb * L

    @pl.when(pl.program_id(0) == 0)
    def _():
        i0 = lax.broadcasted_iota(jnp.int32, (rows, bb), 0)
        i1 = lax.broadcasted_iota(jnp.int32, (rows, bb), 1)
        e_ref[...] = (i0 // L == i1).astype(jnp.float32)
        wtc = wwt_ref[...]
        wtc_ref[...] = wtc - jnp.mean(wtc, axis=1, keepdims=True)
        p = jnp.dot(tt_ref[...], wtt_ref[...],
                    preferred_element_type=jnp.float32)
        pc_ref[...] = p - jnp.mean(p, axis=1, keepdims=True)
        bc = bw_ref[...] + bt_ref[...]
        bcc_ref[...] = bc - jnp.mean(bc, axis=1, keepdims=True)

    wtc = wtc_ref[...]

    tid = tid_ref[...]
    iota = lax.broadcasted_iota(jnp.int32, (bb, T), 1)
    oh = ((iota == tid) & (tid != 0)).astype(jnp.float32)
    te = (jnp.dot(oh, pc_ref[...], preferred_element_type=jnp.float32)
          + bcc_ref[...])

    mask = (wid_ref[...] != 0).astype(jnp.float32)
    xc = (jnp.dot(we_ref[...] * mask, wtc, preferred_element_type=jnp.float32)
          + jnp.dot(e_ref[...], te, preferred_element_type=jnp.float32))
    var = jnp.mean(xc * xc, axis=1, keepdims=True)
    y = xc * lax.rsqrt(var + 1e-5)
    y = y * g_ref[...] + b_ref[...]
    out_ref[...] = y.reshape(bb, L, H)


def _tc_fused(we2, wid2, tid2, topic_table, w_topic_t, w_word_t,
              b_word, b_topic, gamma, beta, init=None, blk0=0, nblk=None,
              bb=128):
    nb = B // bb if nblk is None else nblk
    full2 = lambda shape: pl.BlockSpec(shape, lambda i: (0, 0))
    in_specs = [
        pl.BlockSpec((bb * L, DW), lambda i: (i, 0)),
        pl.BlockSpec((bb * L, 1), lambda i: (i, 0)),
        pl.BlockSpec((bb, 1), lambda i: (i, 0)),
        full2((T, DT)),
        full2((DT, H)),
        full2((DW, H)),
        full2((1, H)),
        full2((1, H)),
        full2((1, H)),
        full2((1, H)),
    ]
    args = [we2, wid2, tid2, topic_table, w_topic_t, w_word_t,
            b_word.reshape(1, H), b_topic.reshape(1, H),
            gamma.reshape(1, H), beta.reshape(1, H)]
    body = _tc_body
    alias = {}
    if init is not None:
        in_specs.append(pl.BlockSpec(memory_space=pl.ANY))
        args.append(init)
        alias = {10: 0}
        body = lambda *r: _tc_body(*r[:10], *r[11:])
    return pl.pallas_call(
        body,
        grid=(nb,),
        in_specs=in_specs,
        out_specs=pl.BlockSpec((bb, L, H), lambda i: (i + blk0, 0, 0)),
        out_shape=jax.ShapeDtypeStruct((B, L, H), jnp.float32),
        scratch_shapes=[pltpu.VMEM((bb * L, bb), jnp.float32),
                        pltpu.VMEM((DW, H), jnp.float32),
                        pltpu.VMEM((T, H), jnp.float32),
                        pltpu.VMEM((1, H), jnp.float32)],
        input_output_aliases=alias,
        compiler_params=pltpu.CompilerParams(
            dimension_semantics=("arbitrary",)),
    )(*args)


def kernel(word_ids, topic_ids, word_table, topic_table, W_word, b_word,
           W_topic, b_topic, gamma, beta):
    ids = word_ids.reshape(N_ROWS)
    wid2 = word_ids.reshape(N_ROWS, 1)
    tid2 = topic_ids.reshape(B, 1)
    common = (topic_table, W_topic.T, W_word.T, b_word, b_topic, gamma, beta)
    h, hb = N_ROWS // 2, B // 2
    we0 = _sc_gather(ids[:h], word_table, h, _CHUNKS_PER_W // 2)
    we1 = _sc_gather(ids[h:], word_table, h, _CHUNKS_PER_W // 2)
    y = _tc_fused(we0, wid2[:h], tid2[:hb], *common, blk0=0, nblk=16)
    y = _tc_fused(we1, wid2[h:], tid2[hb:], *common, init=y, blk0=16, nblk=16)
    return y


# speedup vs baseline: 1.0044x; 1.0044x over previous
import functools

import jax
import jax.numpy as jnp
from jax import lax
from jax.experimental import pallas as pl
from jax.experimental.pallas import tpu as pltpu
from jax.experimental.pallas import tpu_sc as plsc

V, DW, T, DT, H = 100000, 128, 512, 64, 256
B, L = 4096, 50
N_ROWS = B * L

_NC, _NS = 2, 16
_NW = _NC * _NS
_CHUNK = 128
_ROWS_PER_W = N_ROWS // _NW
_CHUNKS_PER_W = _ROWS_PER_W // _CHUNK


def _sc_gather_body(chunks_per_w, ids_hbm, table_hbm, out_hbm, idx_all, rows, sem):
    wid = lax.axis_index("s") * _NC + lax.axis_index("c")
    chunk_base = wid * chunks_per_w
    pltpu.sync_copy(ids_hbm.at[wid], idx_all)

    def chunk(j, carry):
        pltpu.async_copy(table_hbm.at[idx_all.at[j]], rows, sem).wait()
        pltpu.sync_copy(rows, out_hbm.at[pl.ds((chunk_base + j) * _CHUNK, _CHUNK)])
        return carry

    lax.fori_loop(0, chunks_per_w, chunk, 0)


def _sc_gather(word_ids_flat, word_table, n_rows, chunks_per_w):
    ids2d = word_ids_flat.reshape(_NW, chunks_per_w, _CHUNK)
    width = word_table.shape[1]
    mesh = plsc.VectorSubcoreMesh(core_axis_name="c", subcore_axis_name="s")
    k = functools.partial(
        pl.kernel,
        mesh=mesh,
        out_type=jax.ShapeDtypeStruct((n_rows, width), word_table.dtype),
        scratch_types=[
            pltpu.VMEM((chunks_per_w, _CHUNK), jnp.int32),
            pltpu.VMEM((_CHUNK, width), word_table.dtype),
            pltpu.SemaphoreType.DMA,
        ],
    )(functools.partial(_sc_gather_body, chunks_per_w))
    return k(ids2d, word_table)


def _tc_body(we_ref, wid_ref, tid_ref, tt_ref, wtt_ref, wwt_ref,
             bw_ref, bt_ref, g_ref, b_ref, out_ref, e_ref,
             wtc_ref, pc_ref, bcc_ref):
    bb = tid_ref.shape[0]
    rows = bb * L

    @pl.when(pl.program_id(0) == 0)
    def _():
        i0 = lax.broadcasted_iota(jnp.int32, (rows, bb), 0)
        i1 = lax.broadcasted_iota(jnp.int32, (rows, bb), 1)
        e_ref[...] = (i0 // L == i1).astype(jnp.float32)
        wtc = wwt_ref[...]
        wtc_ref[...] = wtc - jnp.mean(wtc, axis=1, keepdims=True)
        p = jnp.dot(tt_ref[...], wtt_ref[...],
                    preferred_element_type=jnp.float32)
        pc_ref[...] = p - jnp.mean(p, axis=1, keepdims=True)
        bc = bw_ref[...] + bt_ref[...]
        bcc_ref[...] = bc - jnp.mean(bc, axis=1, keepdims=True)

    tid = tid_ref[...]
    iota = lax.broadcasted_iota(jnp.int32, (bb, T), 1)
    oh = ((iota == tid) & (tid != 0)).astype(jnp.float32)
    te = (jnp.dot(oh, pc_ref[...], preferred_element_type=jnp.float32)
          + bcc_ref[...])

    mask = (wid_ref[...] != 0).astype(jnp.float32)
    xc = (jnp.dot(we_ref[...] * mask, wtc_ref[...],
                  preferred_element_type=jnp.float32)
          + jnp.dot(e_ref[...], te, preferred_element_type=jnp.float32))
    var = jnp.mean(xc * xc, axis=1, keepdims=True)
    y = xc * lax.rsqrt(var + 1e-5)
    y = y * g_ref[...] + b_ref[...]
    out_ref[...] = y.reshape(bb, L, H)


def _tc_fused(we2, wid2, tid2, topic_table, w_topic_t, w_word_t,
              b_word, b_topic, gamma, beta, init=None, blk0=0, nblk=None,
              bb=128):
    nb = B // bb if nblk is None else nblk
    full2 = lambda shape: pl.BlockSpec(shape, lambda i: (0, 0))
    in_specs = [
        pl.BlockSpec((bb * L, DW), lambda i: (i, 0)),
        pl.BlockSpec((bb * L, 1), lambda i: (i, 0)),
        pl.BlockSpec((bb, 1), lambda i: (i, 0)),
        full2((T, DT)),
        full2((DT, H)),
        full2((DW, H)),
        full2((1, H)),
        full2((1, H)),
        full2((1, H)),
        full2((1, H)),
    ]
    args = [we2, wid2, tid2, topic_table, w_topic_t, w_word_t,
            b_word.reshape(1, H), b_topic.reshape(1, H),
            gamma.reshape(1, H), beta.reshape(1, H)]
    body = _tc_body
    alias = {}
    if init is not None:
        in_specs.append(pl.BlockSpec(memory_space=pl.ANY))
        args.append(init)
        alias = {10: 0}
        body = lambda *r: _tc_body(*r[:10], *r[11:])
    return pl.pallas_call(
        body,
        grid=(nb,),
        in_specs=in_specs,
        out_specs=pl.BlockSpec((bb, L, H), lambda i: (i + blk0, 0, 0)),
        out_shape=jax.ShapeDtypeStruct((B, L, H), jnp.float32),
        scratch_shapes=[pltpu.VMEM((bb * L, bb), jnp.float32),
                        pltpu.VMEM((DW, H), jnp.float32),
                        pltpu.VMEM((T, H), jnp.float32),
                        pltpu.VMEM((1, H), jnp.float32)],
        input_output_aliases=alias,
        compiler_params=pltpu.CompilerParams(
            dimension_semantics=("arbitrary",)),
    )(*args)


def kernel(word_ids, topic_ids, word_table, topic_table, W_word, b_word,
           W_topic, b_topic, gamma, beta):
    ids = word_ids.reshape(N_ROWS)
    wid2 = word_ids.reshape(N_ROWS, 1)
    tid2 = topic_ids.reshape(B, 1)
    h = N_ROWS // 2
    hb = B // 2
    cpw = h // _NW // _CHUNK
    common = (topic_table, W_topic.T, W_word.T, b_word, b_topic, gamma, beta)
    we0 = _sc_gather(ids[:h], word_table, h, cpw)
    we1 = _sc_gather(ids[h:], word_table, h, cpw)
    y = _tc_fused(we0, wid2[:h], tid2[:hb], *common, blk0=0, nblk=16)
    y = _tc_fused(we1, wid2[h:], tid2[hb:], *common, init=y, blk0=16, nblk=16)
    return y


# speedup vs baseline: 1.1461x; 1.1411x over previous
import functools

import jax
import jax.numpy as jnp
from jax import lax
from jax.experimental import pallas as pl
from jax.experimental.pallas import tpu as pltpu
from jax.experimental.pallas import tpu_sc as plsc

V, DW, T, DT, H = 100000, 128, 512, 64, 256
B, L = 4096, 50
N_ROWS = B * L

_NC, _NS = 2, 16
_NW = _NC * _NS
_CHUNK = 128
_ROWS_PER_W = N_ROWS // _NW
_CHUNKS_PER_W = _ROWS_PER_W // _CHUNK


def _sc_gather_body(chunks_per_w, ids_hbm, table_hbm, out_hbm, idx_all,
                    rows0, rows1, gsem0, gsem1, wsem0, wsem1):
    wid = lax.axis_index("s") * _NC + lax.axis_index("c")
    chunk_base = wid * chunks_per_w
    pltpu.sync_copy(ids_hbm.at[wid], idx_all)

    def out_at(j):
        return out_hbm.at[pl.ds((chunk_base + j) * _CHUNK, _CHUNK)]

    def pair(i, carry):
        j0 = 2 * i
        j1 = j0 + 1
        g0 = pltpu.async_copy(table_hbm.at[idx_all.at[j0]], rows0, gsem0)
        g1 = pltpu.async_copy(table_hbm.at[idx_all.at[j1]], rows1, gsem1)
        g0.wait()
        w0 = pltpu.async_copy(rows0, out_at(j0), wsem0)
        g1.wait()
        w1 = pltpu.async_copy(rows1, out_at(j1), wsem1)
        w0.wait()
        w1.wait()
        return carry

    lax.fori_loop(0, chunks_per_w // 2, pair, 0)


def _sc_gather(word_ids_flat, word_table, n_rows, chunks_per_w):
    ids2d = word_ids_flat.reshape(_NW, chunks_per_w, _CHUNK)
    width = word_table.shape[1]
    mesh = plsc.VectorSubcoreMesh(core_axis_name="c", subcore_axis_name="s")
    k = functools.partial(
        pl.kernel,
        mesh=mesh,
        out_type=jax.ShapeDtypeStruct((n_rows, width), word_table.dtype),
        scratch_types=[
            pltpu.VMEM((chunks_per_w, _CHUNK), jnp.int32),
            pltpu.VMEM((_CHUNK, width), word_table.dtype),
            pltpu.VMEM((_CHUNK, width), word_table.dtype),
            pltpu.SemaphoreType.DMA,
            pltpu.SemaphoreType.DMA,
            pltpu.SemaphoreType.DMA,
            pltpu.SemaphoreType.DMA,
        ],
    )(functools.partial(_sc_gather_body, chunks_per_w))
    return k(ids2d, word_table)


def _tc_body(we_ref, wid_ref, tid_ref, tt_ref, wtt_ref, wwt_ref,
             bw_ref, bt_ref, g_ref, b_ref, out_ref, e_ref,
             wtc_ref, pc_ref, bcc_ref):
    bb = tid_ref.shape[0]
    rows = bb * L

    @pl.when(pl.program_id(0) == 0)
    def _():
        i0 = lax.broadcasted_iota(jnp.int32, (rows, bb), 0)
        i1 = lax.broadcasted_iota(jnp.int32, (rows, bb), 1)
        e_ref[...] = (i0 // L == i1).astype(jnp.float32)
        wtc = wwt_ref[...]
        wtc_ref[...] = wtc - jnp.mean(wtc, axis=1, keepdims=True)
        p = jnp.dot(tt_ref[...], wtt_ref[...],
                    preferred_element_type=jnp.float32)
        pc_ref[...] = p - jnp.mean(p, axis=1, keepdims=True)
        bc = bw_ref[...] + bt_ref[...]
        bcc_ref[...] = bc - jnp.mean(bc, axis=1, keepdims=True)

    tid = tid_ref[...]
    iota = lax.broadcasted_iota(jnp.int32, (bb, T), 1)
    oh = ((iota == tid) & (tid != 0)).astype(jnp.float32)
    te = (jnp.dot(oh, pc_ref[...], preferred_element_type=jnp.float32)
          + bcc_ref[...])

    mask = (wid_ref[...] != 0).astype(jnp.float32)
    xc = (jnp.dot(we_ref[...] * mask, wtc_ref[...],
                  preferred_element_type=jnp.float32)
          + jnp.dot(e_ref[...], te, preferred_element_type=jnp.float32))
    var = jnp.mean(xc * xc, axis=1, keepdims=True)
    y = xc * lax.rsqrt(var + 1e-5)
    y = y * g_ref[...] + b_ref[...]
    out_ref[...] = y.reshape(bb, L, H)


def _tc_fused(we2, wid2, tid2, topic_table, w_topic_t, w_word_t,
              b_word, b_topic, gamma, beta, init=None, blk0=0, nblk=None,
              bb=128):
    nb = B // bb if nblk is None else nblk
    full2 = lambda shape: pl.BlockSpec(shape, lambda i: (0, 0))
    in_specs = [
        pl.BlockSpec((bb * L, DW), lambda i: (i, 0)),
        pl.BlockSpec((bb * L, 1), lambda i: (i, 0)),
        pl.BlockSpec((bb, 1), lambda i: (i, 0)),
        full2((T, DT)),
        full2((DT, H)),
        full2((DW, H)),
        full2((1, H)),
        full2((1, H)),
        full2((1, H)),
        full2((1, H)),
    ]
    args = [we2, wid2, tid2, topic_table, w_topic_t, w_word_t,
            b_word.reshape(1, H), b_topic.reshape(1, H),
            gamma.reshape(1, H), beta.reshape(1, H)]
    body = _tc_body
    alias = {}
    if init is not None:
        in_specs.append(pl.BlockSpec(memory_space=pl.ANY))
        args.append(init)
        alias = {10: 0}
        body = lambda *r: _tc_body(*r[:10], *r[11:])
    return pl.pallas_call(
        body,
        grid=(nb,),
        in_specs=in_specs,
        out_specs=pl.BlockSpec((bb, L, H), lambda i: (i + blk0, 0, 0)),
        out_shape=jax.ShapeDtypeStruct((B, L, H), jnp.float32),
        scratch_shapes=[pltpu.VMEM((bb * L, bb), jnp.float32),
                        pltpu.VMEM((DW, H), jnp.float32),
                        pltpu.VMEM((T, H), jnp.float32),
                        pltpu.VMEM((1, H), jnp.float32)],
        input_output_aliases=alias,
        compiler_params=pltpu.CompilerParams(
            dimension_semantics=("arbitrary",)),
    )(*args)


def kernel(word_ids, topic_ids, word_table, topic_table, W_word, b_word,
           W_topic, b_topic, gamma, beta):
    we_flat = _sc_gather(word_ids.reshape(N_ROWS), word_table,
                         N_ROWS, _CHUNKS_PER_W)
    return _tc_fused(we_flat, word_ids.reshape(N_ROWS, 1),
                     topic_ids.reshape(B, 1), topic_table,
                     W_topic.T, W_word.T, b_word, b_topic, gamma, beta)
